# baseline (speedup 1.0000x reference)
import jax
import jax.numpy as jnp
from jax import lax
from jax.experimental import pallas as pl
from jax.experimental.pallas import tpu as pltpu
from jax.experimental.pallas import tpu_sc as plsc

_LAMBDA_POINT = 20.0

_B, _T, _H, _W = 8, 16, 256, 256
_N = 512
_BT = _B * _T
_NC, _NS, _L = 2, 16, 16
_NW = _NC * _NS
_BT_PER_W = _BT // _NW
_PTS_PER_W = _BT_PER_W * _N
_CHUNK = 128
_NCHUNK = _PTS_PER_W // _CHUNK


def _point_loss_sc(pred_hbm, rows_hbm, cols_hbm, sv_hbm, out_hbm,
                   rows_v, cols_v, idx_v, g_v, sv_v, acc_v, sem):
    cid = lax.axis_index("c")
    sid = lax.axis_index("s")
    wid = cid * _NS + sid
    base_bt = wid * _BT_PER_W

    pltpu.sync_copy(rows_hbm, rows_v)
    pltpu.sync_copy(cols_hbm, cols_v)
    pltpu.sync_copy(sv_hbm.at[pl.ds(wid * _PTS_PER_W, _PTS_PER_W)], sv_v)

    n_pt_chunks = _N // _L

    def build(i, carry):
        sl = pl.ds((i % n_pt_chunks) * _L, _L)
        off = (base_bt + i // n_pt_chunks) * (_H * _W)
        idx_v[pl.ds(i * _L, _L)] = rows_v[sl] * _W + cols_v[sl] + off
        return carry

    lax.fori_loop(0, _PTS_PER_W // _L, build, 0)

    copies = []
    for j in range(_NCHUNK):
        sl = pl.ds(j * _CHUNK, _CHUNK)
        copies.append(
            pltpu.async_copy(pred_hbm.at[idx_v.at[sl]], g_v.at[sl], sem))
    for c in copies:
        c.wait()

    def accum(i, acc):
        sl = pl.ds(i * _L, _L)
        d = g_v[sl] - sv_v[sl]
        return acc + d * d

    acc = lax.fori_loop(0, _PTS_PER_W // _L, accum,
                        jnp.zeros((_L,), jnp.float32))
    acc_v[...] = acc * (_LAMBDA_POINT / (_BT * _N))
    pltpu.sync_copy(acc_v, out_hbm.at[wid])


@jax.jit
def kernel(pred, s_coords, s_values):
    pred_flat = pred.reshape(-1)
    rows = s_coords[:, 0].astype(jnp.int32)
    cols = s_coords[:, 1].astype(jnp.int32)
    sv = s_values.reshape(-1).astype(jnp.float32)

    mesh = plsc.VectorSubcoreMesh(core_axis_name="c", subcore_axis_name="s")
    f = pl.kernel(
        _point_loss_sc,
        mesh=mesh,
        out_type=jax.ShapeDtypeStruct((_NW, _L), jnp.float32),
        scratch_types=[
            pltpu.VMEM((_N,), jnp.int32),
            pltpu.VMEM((_N,), jnp.int32),
            pltpu.VMEM((_PTS_PER_W,), jnp.int32),
            pltpu.VMEM((_PTS_PER_W,), jnp.float32),
            pltpu.VMEM((_PTS_PER_W,), jnp.float32),
            pltpu.VMEM((_L,), jnp.float32),
            pltpu.SemaphoreType.DMA,
        ],
    )
    partial = f(pred_flat, rows, cols, sv)
    return jnp.sum(partial)


# speedup vs baseline: 1.4634x; 1.4634x over previous
import jax
import jax.numpy as jnp
from jax import lax
from jax.experimental import pallas as pl
from jax.experimental.pallas import tpu as pltpu
from jax.experimental.pallas import tpu_sc as plsc

_LAMBDA_POINT = 20.0

_B, _T, _H, _W = 8, 16, 256, 256
_N = 512
_BT = _B * _T
_NC, _NS, _L = 2, 16, 16
_NW = _NC * _NS
_BT_PER_W = _BT // _NW
_PTS_PER_W = _BT_PER_W * _N


def _point_loss_sc(pred_hbm, rows_hbm, cols_hbm, sv_hbm, out_hbm,
                   rows_v, cols_v, img_v, sv_v, acc_v, sem):
    cid = lax.axis_index("c")
    sid = lax.axis_index("s")
    wid = cid * _NS + sid

    pltpu.sync_copy(rows_hbm, rows_v)
    pltpu.sync_copy(cols_hbm, cols_v)
    pltpu.sync_copy(sv_hbm.at[pl.ds(wid * _PTS_PER_W, _PTS_PER_W)], sv_v)

    acc = jnp.zeros((_L,), jnp.float32)
    for img in range(_BT_PER_W):
        row0 = (wid * _BT_PER_W + img) * _H
        pltpu.async_copy(pred_hbm.at[pl.ds(row0, _H), :], img_v, sem).wait()

        def chunk(i, a, img=img):
            sl = pl.ds(i * _L, _L)
            g = plsc.load_gather(img_v, [rows_v[sl], cols_v[sl]])
            d = g - sv_v[pl.ds(img * _N + i * _L, _L)]
            return a + d * d

        acc = lax.fori_loop(0, _N // _L, chunk, acc)

    acc_v[...] = acc * (_LAMBDA_POINT / (_BT * _N))
    pltpu.sync_copy(acc_v, out_hbm.at[wid])


@jax.jit
def kernel(pred, s_coords, s_values):
    pred2d = pred.reshape(_BT * _H, _W)
    rows = s_coords[:, 0].astype(jnp.int32)
    cols = s_coords[:, 1].astype(jnp.int32)
    sv = s_values.reshape(-1).astype(jnp.float32)

    mesh = plsc.VectorSubcoreMesh(core_axis_name="c", subcore_axis_name="s")
    f = pl.kernel(
        _point_loss_sc,
        mesh=mesh,
        out_type=jax.ShapeDtypeStruct((_NW, _L), jnp.float32),
        compiler_params=pltpu.CompilerParams(needs_layout_passes=False),
        scratch_types=[
            pltpu.VMEM((_N,), jnp.int32),
            pltpu.VMEM((_N,), jnp.int32),
            pltpu.VMEM((_H, _W), jnp.float32),
            pltpu.VMEM((_PTS_PER_W,), jnp.float32),
            pltpu.VMEM((_L,), jnp.float32),
            pltpu.SemaphoreType.DMA,
        ],
    )
    partial = f(pred2d, rows, cols, sv)
    return jnp.sum(partial)
